# baseline (speedup 1.0000x reference)
import functools

import jax
import jax.numpy as jnp
from jax import lax
from jax.experimental import pallas as pl
from jax.experimental.pallas import tpu as pltpu
from jax.experimental.pallas import tpu_sc as plsc

B = 16
MAX_LEN = 2048
D = 1024
HALF = MAX_LEN // 2

NC = 2
NS = 16

C = 48
NBUF = 2
NCH = -(-HALF // C)
NPBUF = 4

_mesh = plsc.VectorSubcoreMesh(core_axis_name="c", subcore_axis_name="s")


@functools.partial(
    pl.kernel,
    mesh=_mesh,
    out_type=[
        jax.ShapeDtypeStruct((B * MAX_LEN, D), jnp.float32),
        jax.ShapeDtypeStruct((B * MAX_LEN,), jnp.int32),
    ],
    scratch_types=(
        [pltpu.VMEM((32,), jnp.int32),
         pltpu.VMEM((HALF,), jnp.int32),
         pltpu.VMEM((C,), jnp.int32)]
        + [pltpu.VMEM((C, D), jnp.float32) for _ in range(NBUF)]
        + [pltpu.VMEM((C,), jnp.int32) for _ in range(NBUF)]
        + [pltpu.VMEM((C,), jnp.int32) for _ in range(NPBUF)]
        + [pltpu.SemaphoreType.DMA for _ in range(2 * NBUF + NPBUF)]
    ),
    compiler_params=pltpu.CompilerParams(needs_layout_passes=False),
)
def _pack(cu_hbm, flat_hbm, zeros_hbm, padded_hbm, mask_hbm,
          cu_v, mask_v, sidx, *rest):
    bufs = rest[:NBUF]
    gidx = rest[NBUF:2 * NBUF]
    pidx = rest[2 * NBUF:2 * NBUF + NPBUF]
    insem = rest[2 * NBUF + NPBUF:3 * NBUF + NPBUF]
    outsem = rest[3 * NBUF + NPBUF:4 * NBUF + NPBUF]
    psem = rest[4 * NBUF + NPBUF:]

    wid = lax.axis_index("c") * NS + lax.axis_index("s")
    b = wid // 2
    p0 = (wid % 2) * HALF

    pltpu.sync_copy(cu_hbm, cu_v)
    lane = lax.iota(jnp.int32, 16)
    sel = lane == b
    start_b = jnp.sum(jnp.where(sel, cu_v[pl.ds(0, 16)], 0))
    len_b = jnp.sum(jnp.where(sel, cu_v[pl.ds(16, 16)], 0))

    n_real = jnp.clip(len_b - p0, 0, HALF)
    n_pad = HALF - n_real
    src0 = start_b + p0
    out0 = b * MAX_LEN + p0
    zbase = out0 + n_real

    nzch = (n_pad + C - 1) // C

    @pl.when(nzch > 0)
    def _pads():
        pltpu.sync_copy(zeros_hbm, bufs[0])

    for j in range(NCH):
        slot = j % NPBUF

        @pl.when(j < nzch)
        def _pad(j=j, slot=slot):
            if j >= NPBUF:
                pltpu.make_async_copy(bufs[0], padded_hbm.at[pidx[slot]],
                                      psem[slot]).wait()
            for h in range(0, C, 16):
                q = jnp.minimum(j * C + h + lane, n_pad - 1)
                pidx[slot][pl.ds(h, 16)] = zbase + q
            pltpu.make_async_copy(bufs[0], padded_hbm.at[pidx[slot]],
                                  psem[slot]).start()

    for s in range(NPBUF):
        @pl.when(nzch > s)
        def _pdrain(s=s):
            pltpu.make_async_copy(bufs[0], padded_hbm.at[pidx[s]],
                                  psem[s]).wait()

    nch = (n_real + C - 1) // C

    def fill_gidx(i, slot):
        for h in range(0, C, 16):
            q = jnp.minimum(i * C + h + lane, n_real - 1)
            gidx[slot][pl.ds(h, 16)] = src0 + q

    for j in range(NBUF):
        @pl.when(j < nch)
        def _prime(j=j):
            fill_gidx(j, j)
            pltpu.make_async_copy(flat_hbm.at[gidx[j]], bufs[j],
                                  insem[j]).start()

    for i in range(NCH):
        slot = i % NBUF

        @pl.when(i < nch)
        def _chunk(i=i, slot=slot):
            pltpu.make_async_copy(flat_hbm.at[gidx[slot]], bufs[slot],
                                  insem[slot]).wait()

            @pl.when((i + 1) * C <= n_real)
            def _full():
                pltpu.make_async_copy(bufs[slot],
                                      padded_hbm.at[pl.ds(out0 + i * C, C)],
                                      outsem[slot]).start()

            @pl.when((i + 1) * C > n_real)
            def _boundary():
                for h in range(0, C, 16):
                    q = jnp.minimum(i * C + h + lane, n_real - 1)
                    sidx[pl.ds(h, 16)] = out0 + q
                pltpu.make_async_copy(bufs[slot], padded_hbm.at[sidx],
                                      outsem[slot]).start()

        @pl.when(i + NBUF < nch)
        def _next(i=i, slot=slot):
            pltpu.make_async_copy(bufs[slot],
                                  padded_hbm.at[pl.ds(out0, C)],
                                  outsem[slot]).wait()
            fill_gidx(i + NBUF, slot)
            pltpu.make_async_copy(flat_hbm.at[gidx[slot]], bufs[slot],
                                  insem[slot]).start()

    def mrow(k, carry):
        mask_v[pl.ds(k * 16, 16)] = (lane + (p0 + k * 16) < len_b).astype(
            jnp.int32)
        return carry

    lax.fori_loop(0, HALF // 16, mrow, 0)
    pltpu.sync_copy(mask_v, mask_hbm.at[pl.ds(out0, HALF)])

    for s in range(NBUF):
        @pl.when(nch > s)
        def _drain(s=s):
            pltpu.make_async_copy(bufs[s], padded_hbm.at[pl.ds(out0, C)],
                                  outsem[s]).wait()


def kernel(flat, cu_seqlens):
    cu32 = jnp.concatenate([cu_seqlens[:B],
                            cu_seqlens[1:] - cu_seqlens[:-1]])
    zeros = jnp.zeros((C, D), jnp.float32)
    padded_flat, mask_flat = _pack(cu32, flat, zeros)
    return (padded_flat.reshape(B, MAX_LEN, D),
            mask_flat.reshape(B, MAX_LEN))


# speedup vs baseline: 1.0342x; 1.0342x over previous
import functools

import jax
import jax.numpy as jnp
from jax import lax
from jax.experimental import pallas as pl
from jax.experimental.pallas import tpu as pltpu
from jax.experimental.pallas import tpu_sc as plsc

B = 16
MAX_LEN = 2048
D = 1024
HALF = MAX_LEN // 2

NC = 2
NS = 16

C = 48
NBUF = 2
NCH = -(-HALF // C)
NPBUF = 4

_mesh = plsc.VectorSubcoreMesh(core_axis_name="c", subcore_axis_name="s")


@functools.partial(
    pl.kernel,
    mesh=_mesh,
    out_type=[
        jax.ShapeDtypeStruct((B * MAX_LEN, D), jnp.float32),
        jax.ShapeDtypeStruct((B * MAX_LEN,), jnp.int32),
    ],
    scratch_types=(
        [pltpu.VMEM((32,), jnp.int32),
         pltpu.VMEM((HALF,), jnp.int32),
         pltpu.VMEM((C,), jnp.int32)]
        + [pltpu.VMEM((C, D), jnp.float32) for _ in range(NBUF)]
        + [pltpu.VMEM((C,), jnp.int32) for _ in range(NBUF)]
        + [pltpu.VMEM((16,), jnp.int32)]
        + [pltpu.SemaphoreType.DMA for _ in range(2 * NBUF + 3)]
    ),
    compiler_params=pltpu.CompilerParams(needs_layout_passes=False),
)
def _pack(cu_hbm, flat_hbm, zeros_hbm, padded_hbm, mask_hbm,
          cu_v, mask_v, sidx, *rest):
    bufs = rest[:NBUF]
    gidx = rest[NBUF:2 * NBUF]
    pidx = rest[2 * NBUF]
    insem = rest[2 * NBUF + 1:3 * NBUF + 1]
    outsem = rest[3 * NBUF + 1:4 * NBUF + 1]
    psem = rest[4 * NBUF + 1:]

    wid = lax.axis_index("c") * NS + lax.axis_index("s")
    b = wid // 2
    p0 = (wid % 2) * HALF

    pltpu.sync_copy(cu_hbm, cu_v)
    lane = lax.iota(jnp.int32, 16)
    sel = lane == b
    start_b = jnp.sum(jnp.where(sel, cu_v[pl.ds(0, 16)], 0))
    len_b = jnp.sum(jnp.where(sel, cu_v[pl.ds(16, 16)], 0))

    n_real = jnp.clip(len_b - p0, 0, HALF)
    n_pad = HALF - n_real
    src0 = start_b + p0
    out0 = b * MAX_LEN + p0
    zbase = out0 + n_real

    end = out0 + HALF
    a0 = pl.multiple_of(zbase + (8 - (zbase % 8)) % 8, 8)
    nlin = end - a0
    nzch = nlin // C

    @pl.when(n_pad > 0)
    def _pads():
        pltpu.sync_copy(zeros_hbm, bufs[0])
        q = jnp.minimum(lane, n_pad - 1)
        pidx[pl.ds(0, 16)] = zbase + q
        pltpu.make_async_copy(bufs[0].at[pl.ds(0, 16)],
                              padded_hbm.at[pidx], psem[0]).start()

    for j in range(NCH):
        slot = j % NPBUF

        @pl.when(j < nzch)
        def _pad(j=j, slot=slot):
            if j >= NPBUF:
                pltpu.make_async_copy(bufs[0],
                                      padded_hbm.at[pl.ds(a0, C)],
                                      psem[1]).wait()
            pltpu.make_async_copy(bufs[0],
                                  padded_hbm.at[pl.ds(end - (j + 1) * C, C)],
                                  psem[1]).start()

    rem_base = a0
    for s in (32, 16, 8):
        @pl.when(((nlin % C) & s) != 0)
        def _prem(s=s, rem_base=rem_base):
            pltpu.make_async_copy(bufs[0].at[pl.ds(0, s)],
                                  padded_hbm.at[pl.ds(pl.multiple_of(rem_base, 8), s)],
                                  psem[2]).start()

        rem_base = rem_base + jnp.where(((nlin % C) & s) != 0, s, 0)

    @pl.when(n_pad > 0)
    def _pdrain0():
        pltpu.make_async_copy(bufs[0].at[pl.ds(0, 16)],
                              padded_hbm.at[pidx], psem[0]).wait()

    def _pdrain(j, carry):
        pltpu.make_async_copy(bufs[0], padded_hbm.at[pl.ds(a0, C)],
                              psem[1]).wait()
        return carry

    lax.fori_loop(0, jnp.minimum(nzch, NPBUF), _pdrain, 0)
    for s in (32, 16, 8):
        @pl.when(((nlin % C) & s) != 0)
        def _premw(s=s):
            pltpu.make_async_copy(bufs[0].at[pl.ds(0, s)],
                                  padded_hbm.at[pl.ds(a0, s)],
                                  psem[2]).wait()

    nch = (n_real + C - 1) // C

    def fill_gidx(i, slot):
        for h in range(0, C, 16):
            q = jnp.minimum(i * C + h + lane, n_real - 1)
            gidx[slot][pl.ds(h, 16)] = src0 + q

    for j in range(NBUF):
        @pl.when(j < nch)
        def _prime(j=j):
            fill_gidx(j, j)
            pltpu.make_async_copy(flat_hbm.at[gidx[j]], bufs[j],
                                  insem[j]).start()

    for i in range(NCH):
        slot = i % NBUF

        @pl.when(i < nch)
        def _chunk(i=i, slot=slot):
            pltpu.make_async_copy(flat_hbm.at[gidx[slot]], bufs[slot],
                                  insem[slot]).wait()

            @pl.when((i + 1) * C <= n_real)
            def _full():
                pltpu.make_async_copy(bufs[slot],
                                      padded_hbm.at[pl.ds(out0 + i * C, C)],
                                      outsem[slot]).start()

            @pl.when((i + 1) * C > n_real)
            def _boundary():
                for h in range(0, C, 16):
                    q = jnp.minimum(i * C + h + lane, n_real - 1)
                    sidx[pl.ds(h, 16)] = out0 + q
                pltpu.make_async_copy(bufs[slot], padded_hbm.at[sidx],
                                      outsem[slot]).start()

        @pl.when(i + NBUF < nch)
        def _next(i=i, slot=slot):
            pltpu.make_async_copy(bufs[slot],
                                  padded_hbm.at[pl.ds(out0, C)],
                                  outsem[slot]).wait()
            fill_gidx(i + NBUF, slot)
            pltpu.make_async_copy(flat_hbm.at[gidx[slot]], bufs[slot],
                                  insem[slot]).start()

    def mrow(k, carry):
        mask_v[pl.ds(k * 16, 16)] = (lane + (p0 + k * 16) < len_b).astype(
            jnp.int32)
        return carry

    lax.fori_loop(0, HALF // 16, mrow, 0)
    pltpu.sync_copy(mask_v, mask_hbm.at[pl.ds(out0, HALF)])

    for s in range(NBUF):
        @pl.when(nch > s)
        def _drain(s=s):
            pltpu.make_async_copy(bufs[s], padded_hbm.at[pl.ds(out0, C)],
                                  outsem[s]).wait()


def kernel(flat, cu_seqlens):
    cu32 = jnp.concatenate([cu_seqlens[:B],
                            cu_seqlens[1:] - cu_seqlens[:-1]])
    zeros = jnp.zeros((C, D), jnp.float32)
    padded_flat, mask_flat = _pack(cu32, flat, zeros)
    return (padded_flat.reshape(B, MAX_LEN, D),
            mask_flat.reshape(B, MAX_LEN))


# speedup vs baseline: 1.0421x; 1.0077x over previous
import functools

import jax
import jax.numpy as jnp
from jax import lax
from jax.experimental import pallas as pl
from jax.experimental.pallas import tpu as pltpu
from jax.experimental.pallas import tpu_sc as plsc

B = 16
MAX_LEN = 2048
D = 1024
TOTAL = B * MAX_LEN // 2
HALF = MAX_LEN // 2

NC = 2
NS = 16
RPW = TOTAL // (NC * NS)

C = 32
NBUF = 2
NCH = RPW // C
NZBUF = 4
PCH = -(-HALF // C)

_mesh = plsc.VectorSubcoreMesh(core_axis_name="c", subcore_axis_name="s")


@functools.partial(
    pl.kernel,
    mesh=_mesh,
    out_type=[
        jax.ShapeDtypeStruct((B * MAX_LEN, D), jnp.float32),
        jax.ShapeDtypeStruct((B * MAX_LEN,), jnp.int32),
    ],
    scratch_types=(
        [pltpu.VMEM((32,), jnp.int32),
         pltpu.VMEM((HALF,), jnp.int32),
         pltpu.VMEM((16,), jnp.int32)]
        + [pltpu.VMEM((C, D), jnp.float32) for _ in range(NBUF)]
        + [pltpu.VMEM((C,), jnp.int32) for _ in range(NBUF)]
        + [pltpu.SemaphoreType.DMA for _ in range(2 * NBUF + 3)]
    ),
    compiler_params=pltpu.CompilerParams(needs_layout_passes=False),
)
def _pack(cu_hbm, flat_hbm, zeros_hbm, padded_hbm, mask_hbm,
          cu_v, mask_v, pidx, *rest):
    bufs = rest[:NBUF]
    didx = rest[NBUF:2 * NBUF]
    insem = rest[2 * NBUF:3 * NBUF]
    outsem = rest[3 * NBUF:4 * NBUF]
    psem = rest[4 * NBUF:]

    wid = lax.axis_index("c") * NS + lax.axis_index("s")
    b = wid // 2
    p0 = (wid % 2) * HALF

    pltpu.sync_copy(cu_hbm, cu_v)
    lane = lax.iota(jnp.int32, 16)
    starts_vec = cu_v[pl.ds(0, 16)]
    ends_vec = starts_vec + cu_v[pl.ds(16, 16)]
    sel = lane == b
    len_b = jnp.sum(jnp.where(sel, cu_v[pl.ds(16, 16)], 0))
    ends = [jnp.sum(jnp.where(lane == j, ends_vec, 0)) for j in range(B)]

    n_real = jnp.clip(len_b - p0, 0, HALF)
    n_pad = HALF - n_real
    out0 = b * MAX_LEN + p0
    zbase = out0 + n_real

    fbase = wid * RPW

    def fill_didx(i, slot):
        for h in (0, 16):
            t = fbase + i * C + h + lane
            bt = jnp.zeros((16,), jnp.int32)
            for j in range(B):
                bt = bt + (t >= ends[j]).astype(jnp.int32)
            s_bt = plsc.load_gather(cu_v, [bt])
            didx[slot][pl.ds(h, 16)] = bt * MAX_LEN + t - s_bt

    for j in range(NBUF):
        pltpu.make_async_copy(flat_hbm.at[pl.ds(fbase + j * C, C)],
                              bufs[j], insem[j]).start()

    for i in range(NCH):
        slot = i % NBUF
        pltpu.make_async_copy(flat_hbm.at[pl.ds(fbase + i * C, C)],
                              bufs[slot], insem[slot]).wait()
        fill_didx(i, slot)
        pltpu.make_async_copy(bufs[slot], padded_hbm.at[didx[slot]],
                              outsem[slot]).start()
        if i + NBUF < NCH:
            pltpu.make_async_copy(bufs[slot], padded_hbm.at[didx[slot]],
                                  outsem[slot]).wait()
            pltpu.make_async_copy(
                flat_hbm.at[pl.ds(fbase + (i + NBUF) * C, C)],
                bufs[slot], insem[slot]).start()

    for s in range(NBUF):
        pltpu.make_async_copy(bufs[s], padded_hbm.at[didx[s]],
                              outsem[s]).wait()

    end = out0 + HALF
    a0 = pl.multiple_of(zbase + (8 - (zbase % 8)) % 8, 8)
    nlin = end - a0
    nzch = nlin // C

    @pl.when(n_pad > 0)
    def _pads():
        pltpu.sync_copy(zeros_hbm, bufs[0])
        q = jnp.minimum(lane, n_pad - 1)
        pidx[pl.ds(0, 16)] = zbase + q
        pltpu.make_async_copy(bufs[0].at[pl.ds(0, 16)],
                              padded_hbm.at[pidx], psem[0]).start()

    for j in range(PCH):
        @pl.when(j < nzch)
        def _pad(j=j):
            if j >= NZBUF:
                pltpu.make_async_copy(bufs[0],
                                      padded_hbm.at[pl.ds(a0, C)],
                                      psem[1]).wait()
            pltpu.make_async_copy(bufs[0],
                                  padded_hbm.at[pl.ds(end - (j + 1) * C, C)],
                                  psem[1]).start()

    rem_base = a0
    for s in (16, 8):
        @pl.when(((nlin % C) & s) != 0)
        def _prem(s=s, rem_base=rem_base):
            pltpu.make_async_copy(
                bufs[0].at[pl.ds(0, s)],
                padded_hbm.at[pl.ds(pl.multiple_of(rem_base, 8), s)],
                psem[2]).start()

        rem_base = rem_base + jnp.where(((nlin % C) & s) != 0, s, 0)

    def mrow(k, carry):
        mask_v[pl.ds(k * 16, 16)] = (lane + (p0 + k * 16) < len_b).astype(
            jnp.int32)
        return carry

    lax.fori_loop(0, HALF // 16, mrow, 0)
    pltpu.sync_copy(mask_v, mask_hbm.at[pl.ds(out0, HALF)])

    @pl.when(n_pad > 0)
    def _pdrain0():
        pltpu.make_async_copy(bufs[0].at[pl.ds(0, 16)],
                              padded_hbm.at[pidx], psem[0]).wait()

    def _pdrain(j, carry):
        pltpu.make_async_copy(bufs[0], padded_hbm.at[pl.ds(a0, C)],
                              psem[1]).wait()
        return carry

    lax.fori_loop(0, jnp.minimum(nzch, NZBUF), _pdrain, 0)
    for s in (16, 8):
        @pl.when(((nlin % C) & s) != 0)
        def _premw(s=s):
            pltpu.make_async_copy(bufs[0].at[pl.ds(0, s)],
                                  padded_hbm.at[pl.ds(a0, s)],
                                  psem[2]).wait()


def kernel(flat, cu_seqlens):
    cu32 = jnp.concatenate([cu_seqlens[:B],
                            cu_seqlens[1:] - cu_seqlens[:-1]])
    zeros = jnp.zeros((C, D), jnp.float32)
    padded_flat, mask_flat = _pack(cu32, flat, zeros)
    return (padded_flat.reshape(B, MAX_LEN, D),
            mask_flat.reshape(B, MAX_LEN))


# speedup vs baseline: 1.0483x; 1.0059x over previous
import functools

import jax
import jax.numpy as jnp
from jax import lax
from jax.experimental import pallas as pl
from jax.experimental.pallas import tpu as pltpu
from jax.experimental.pallas import tpu_sc as plsc

B = 16
MAX_LEN = 2048
D = 1024
TOTAL = B * MAX_LEN // 2
HALF = MAX_LEN // 2

NC = 2
NS = 16
RPW = TOTAL // (NC * NS)

C = 32
NBUF = 3
NCH = RPW // C
NZBUF = 4
PCH = -(-HALF // C)

_mesh = plsc.VectorSubcoreMesh(core_axis_name="c", subcore_axis_name="s")


@functools.partial(
    pl.kernel,
    mesh=_mesh,
    out_type=[
        jax.ShapeDtypeStruct((B * MAX_LEN, D), jnp.float32),
        jax.ShapeDtypeStruct((B * MAX_LEN,), jnp.int32),
    ],
    scratch_types=(
        [pltpu.VMEM((32,), jnp.int32),
         pltpu.VMEM((HALF,), jnp.int32),
         pltpu.VMEM((16,), jnp.int32)]
        + [pltpu.VMEM((C, D), jnp.float32) for _ in range(NBUF)]
        + [pltpu.VMEM((C,), jnp.int32) for _ in range(NBUF)]
        + [pltpu.SemaphoreType.DMA for _ in range(2 * NBUF + 3)]
    ),
    compiler_params=pltpu.CompilerParams(needs_layout_passes=False),
)
def _pack(cu_hbm, flat_hbm, zeros_hbm, padded_hbm, mask_hbm,
          cu_v, mask_v, pidx, *rest):
    bufs = rest[:NBUF]
    didx = rest[NBUF:2 * NBUF]
    insem = rest[2 * NBUF:3 * NBUF]
    outsem = rest[3 * NBUF:4 * NBUF]
    psem = rest[4 * NBUF:]

    wid = lax.axis_index("c") * NS + lax.axis_index("s")
    b = wid // 2
    p0 = (wid % 2) * HALF

    pltpu.sync_copy(cu_hbm, cu_v)
    lane = lax.iota(jnp.int32, 16)
    starts_vec = cu_v[pl.ds(0, 16)]
    ends_vec = starts_vec + cu_v[pl.ds(16, 16)]
    sel = lane == b
    len_b = jnp.sum(jnp.where(sel, cu_v[pl.ds(16, 16)], 0))
    ends = [jnp.sum(jnp.where(lane == j, ends_vec, 0)) for j in range(B)]

    n_real = jnp.clip(len_b - p0, 0, HALF)
    n_pad = HALF - n_real
    out0 = b * MAX_LEN + p0
    zbase = out0 + n_real

    fbase = wid * RPW

    def fill_didx(i, slot):
        for h in (0, 16):
            t = fbase + i * C + h + lane
            bt = jnp.zeros((16,), jnp.int32)
            for j in range(B):
                bt = bt + (t >= ends[j]).astype(jnp.int32)
            s_bt = plsc.load_gather(cu_v, [bt])
            didx[slot][pl.ds(h, 16)] = bt * MAX_LEN + t - s_bt

    for j in range(NBUF):
        pltpu.make_async_copy(flat_hbm.at[pl.ds(fbase + j * C, C)],
                              bufs[j], insem[j]).start()

    for i in range(NCH):
        slot = i % NBUF
        pltpu.make_async_copy(flat_hbm.at[pl.ds(fbase + i * C, C)],
                              bufs[slot], insem[slot]).wait()
        fill_didx(i, slot)
        pltpu.make_async_copy(bufs[slot], padded_hbm.at[didx[slot]],
                              outsem[slot]).start()
        if i + NBUF < NCH:
            pltpu.make_async_copy(bufs[slot], padded_hbm.at[didx[slot]],
                                  outsem[slot]).wait()
            pltpu.make_async_copy(
                flat_hbm.at[pl.ds(fbase + (i + NBUF) * C, C)],
                bufs[slot], insem[slot]).start()

    for s in range(NBUF):
        pltpu.make_async_copy(bufs[s], padded_hbm.at[didx[s]],
                              outsem[s]).wait()

    end = out0 + HALF
    a0 = pl.multiple_of(zbase + (8 - (zbase % 8)) % 8, 8)
    nlin = end - a0
    nzch = nlin // C

    @pl.when(n_pad > 0)
    def _pads():
        pltpu.sync_copy(zeros_hbm, bufs[0])
        q = jnp.minimum(lane, n_pad - 1)
        pidx[pl.ds(0, 16)] = zbase + q
        pltpu.make_async_copy(bufs[0].at[pl.ds(0, 16)],
                              padded_hbm.at[pidx], psem[0]).start()

    for j in range(PCH):
        @pl.when(j < nzch)
        def _pad(j=j):
            if j >= NZBUF:
                pltpu.make_async_copy(bufs[0],
                                      padded_hbm.at[pl.ds(a0, C)],
                                      psem[1]).wait()
            pltpu.make_async_copy(bufs[0],
                                  padded_hbm.at[pl.ds(end - (j + 1) * C, C)],
                                  psem[1]).start()

    rem_base = a0
    for s in (16, 8):
        @pl.when(((nlin % C) & s) != 0)
        def _prem(s=s, rem_base=rem_base):
            pltpu.make_async_copy(
                bufs[0].at[pl.ds(0, s)],
                padded_hbm.at[pl.ds(pl.multiple_of(rem_base, 8), s)],
                psem[2]).start()

        rem_base = rem_base + jnp.where(((nlin % C) & s) != 0, s, 0)

    def mrow(k, carry):
        mask_v[pl.ds(k * 16, 16)] = (lane + (p0 + k * 16) < len_b).astype(
            jnp.int32)
        return carry

    lax.fori_loop(0, HALF // 16, mrow, 0)
    pltpu.sync_copy(mask_v, mask_hbm.at[pl.ds(out0, HALF)])

    @pl.when(n_pad > 0)
    def _pdrain0():
        pltpu.make_async_copy(bufs[0].at[pl.ds(0, 16)],
                              padded_hbm.at[pidx], psem[0]).wait()

    def _pdrain(j, carry):
        pltpu.make_async_copy(bufs[0], padded_hbm.at[pl.ds(a0, C)],
                              psem[1]).wait()
        return carry

    lax.fori_loop(0, jnp.minimum(nzch, NZBUF), _pdrain, 0)
    for s in (16, 8):
        @pl.when(((nlin % C) & s) != 0)
        def _premw(s=s):
            pltpu.make_async_copy(bufs[0].at[pl.ds(0, s)],
                                  padded_hbm.at[pl.ds(a0, s)],
                                  psem[2]).wait()


def kernel(flat, cu_seqlens):
    cu32 = jnp.concatenate([cu_seqlens[:B],
                            cu_seqlens[1:] - cu_seqlens[:-1]])
    zeros = jnp.zeros((C, D), jnp.float32)
    padded_flat, mask_flat = _pack(cu32, flat, zeros)
    return (padded_flat.reshape(B, MAX_LEN, D),
            mask_flat.reshape(B, MAX_LEN))


# speedup vs baseline: 1.0585x; 1.0097x over previous
import functools

import jax
import jax.numpy as jnp
from jax import lax
from jax.experimental import pallas as pl
from jax.experimental.pallas import tpu as pltpu
from jax.experimental.pallas import tpu_sc as plsc

B = 16
MAX_LEN = 2048
D = 1024
TOTAL = B * MAX_LEN // 2
HALF = MAX_LEN // 2

NC = 2
NS = 16
RPW = TOTAL // (NC * NS)

C = 16
NBUF = 6
DLY = 3
NCH = RPW // C
NZBUF = 4
PCH = -(-HALF // C)

_mesh = plsc.VectorSubcoreMesh(core_axis_name="c", subcore_axis_name="s")


@functools.partial(
    pl.kernel,
    mesh=_mesh,
    out_type=[
        jax.ShapeDtypeStruct((B * MAX_LEN, D), jnp.float32),
        jax.ShapeDtypeStruct((B * MAX_LEN,), jnp.int32),
    ],
    scratch_types=(
        [pltpu.VMEM((32,), jnp.int32),
         pltpu.VMEM((HALF,), jnp.int32),
         pltpu.VMEM((16,), jnp.int32)]
        + [pltpu.VMEM((C, D), jnp.float32) for _ in range(NBUF)]
        + [pltpu.VMEM((C,), jnp.int32) for _ in range(NBUF)]
        + [pltpu.SemaphoreType.DMA for _ in range(2 * NBUF + 3)]
    ),
    compiler_params=pltpu.CompilerParams(needs_layout_passes=False),
)
def _pack(cu_hbm, flat_hbm, zeros_hbm, padded_hbm, mask_hbm,
          cu_v, mask_v, pidx, *rest):
    bufs = rest[:NBUF]
    didx = rest[NBUF:2 * NBUF]
    insem = rest[2 * NBUF:3 * NBUF]
    outsem = rest[3 * NBUF:4 * NBUF]
    psem = rest[4 * NBUF:]

    wid = lax.axis_index("c") * NS + lax.axis_index("s")
    b = wid // 2
    p0 = (wid % 2) * HALF

    pltpu.sync_copy(cu_hbm, cu_v)
    lane = lax.iota(jnp.int32, 16)
    starts_vec = cu_v[pl.ds(0, 16)]
    ends_vec = starts_vec + cu_v[pl.ds(16, 16)]
    sel = lane == b
    len_b = jnp.sum(jnp.where(sel, cu_v[pl.ds(16, 16)], 0))
    ends = [jnp.sum(jnp.where(lane == j, ends_vec, 0)) for j in range(B)]

    n_real = jnp.clip(len_b - p0, 0, HALF)
    n_pad = HALF - n_real
    out0 = b * MAX_LEN + p0
    zbase = out0 + n_real

    fbase = wid * RPW

    def fill_didx(i, slot):
        for h in range(0, C, 16):
            t = fbase + i * C + h + lane
            bt = jnp.zeros((16,), jnp.int32)
            for j in range(B):
                bt = bt + (t >= ends[j]).astype(jnp.int32)
            s_bt = plsc.load_gather(cu_v, [bt])
            didx[slot][pl.ds(h, 16)] = bt * MAX_LEN + t - s_bt

    for j in range(NBUF):
        pltpu.make_async_copy(flat_hbm.at[pl.ds(fbase + j * C, C)],
                              bufs[j], insem[j]).start()

    for i in range(NCH):
        slot = i % NBUF
        g = i + NBUF - DLY
        if i - DLY >= 0 and g < NCH:
            gslot = g % NBUF
            pltpu.make_async_copy(bufs[gslot], padded_hbm.at[didx[gslot]],
                                  outsem[gslot]).wait()
            pltpu.make_async_copy(flat_hbm.at[pl.ds(fbase + g * C, C)],
                                  bufs[gslot], insem[gslot]).start()
        pltpu.make_async_copy(flat_hbm.at[pl.ds(fbase + i * C, C)],
                              bufs[slot], insem[slot]).wait()
        fill_didx(i, slot)
        pltpu.make_async_copy(bufs[slot], padded_hbm.at[didx[slot]],
                              outsem[slot]).start()

    for s in range(NBUF):
        pltpu.make_async_copy(bufs[s], padded_hbm.at[didx[s]],
                              outsem[s]).wait()

    end = out0 + HALF
    a0 = pl.multiple_of(zbase + (8 - (zbase % 8)) % 8, 8)
    nlin = end - a0
    nzch = nlin // C

    @pl.when(n_pad > 0)
    def _pads():
        pltpu.sync_copy(zeros_hbm, bufs[0])
        q = jnp.minimum(lane, n_pad - 1)
        pidx[pl.ds(0, 16)] = zbase + q
        pltpu.make_async_copy(bufs[0].at[pl.ds(0, 16)],
                              padded_hbm.at[pidx], psem[0]).start()

    for j in range(PCH):
        @pl.when(j < nzch)
        def _pad(j=j):
            if j >= NZBUF:
                pltpu.make_async_copy(bufs[0],
                                      padded_hbm.at[pl.ds(a0, C)],
                                      psem[1]).wait()
            pltpu.make_async_copy(bufs[0],
                                  padded_hbm.at[pl.ds(end - (j + 1) * C, C)],
                                  psem[1]).start()

    rem_base = a0
    for s in (16, 8):
        @pl.when(((nlin % C) & s) != 0)
        def _prem(s=s, rem_base=rem_base):
            pltpu.make_async_copy(
                bufs[0].at[pl.ds(0, s)],
                padded_hbm.at[pl.ds(pl.multiple_of(rem_base, 8), s)],
                psem[2]).start()

        rem_base = rem_base + jnp.where(((nlin % C) & s) != 0, s, 0)

    def mrow(k, carry):
        mask_v[pl.ds(k * 16, 16)] = (lane + (p0 + k * 16) < len_b).astype(
            jnp.int32)
        return carry

    lax.fori_loop(0, HALF // 16, mrow, 0)
    pltpu.sync_copy(mask_v, mask_hbm.at[pl.ds(out0, HALF)])

    @pl.when(n_pad > 0)
    def _pdrain0():
        pltpu.make_async_copy(bufs[0].at[pl.ds(0, 16)],
                              padded_hbm.at[pidx], psem[0]).wait()

    def _pdrain(j, carry):
        pltpu.make_async_copy(bufs[0], padded_hbm.at[pl.ds(a0, C)],
                              psem[1]).wait()
        return carry

    lax.fori_loop(0, jnp.minimum(nzch, NZBUF), _pdrain, 0)
    for s in (16, 8):
        @pl.when(((nlin % C) & s) != 0)
        def _premw(s=s):
            pltpu.make_async_copy(bufs[0].at[pl.ds(0, s)],
                                  padded_hbm.at[pl.ds(a0, s)],
                                  psem[2]).wait()


def kernel(flat, cu_seqlens):
    cu32 = jnp.concatenate([cu_seqlens[:B],
                            cu_seqlens[1:] - cu_seqlens[:-1]])
    zeros = jnp.zeros((C, D), jnp.float32)
    padded_flat, mask_flat = _pack(cu32, flat, zeros)
    return (padded_flat.reshape(B, MAX_LEN, D),
            mask_flat.reshape(B, MAX_LEN))
